# baseline (device time: 234635 ns/iter reference)
import jax
import jax.numpy as jnp
from jax import lax
from jax.experimental import pallas as pl
from jax.experimental.pallas import tpu as pltpu

C = 512


def kernel(Q, K, V):
    B, Sq, H, D = Q.shape
    Skv = K.shape[1]
    BH = B * H
    NC = Skv // C
    scale = D ** -0.5

    def body(q_ref, k_ref, v_ref, out_ref,
             acc_o, o_comm, l_comm, send_sems, recv_sems):
        b = pl.program_id(0)
        c = pl.program_id(1)
        x = lax.axis_index("x")
        y = lax.axis_index("y")
        z = lax.axis_index("z")
        peer = (x, y, 1 - z)

        @pl.when(jnp.logical_and(b == 0, c == 0))
        def _entry_barrier():
            bsem = pltpu.get_barrier_semaphore()
            pl.semaphore_signal(
                bsem, inc=1, device_id=peer, device_id_type=pl.DeviceIdType.MESH
            )
            pl.semaphore_wait(bsem, 1)

        for h in range(H):
            q = q_ref[0, :, h * D:(h + 1) * D].astype(jnp.bfloat16)
            k = k_ref[0, :, h * D:(h + 1) * D].astype(jnp.bfloat16)
            v = v_ref[0, :, h * D:(h + 1) * D].astype(jnp.bfloat16)
            s = lax.dot_general(
                q, k, (((1,), (1,)), ((), ())), preferred_element_type=jnp.float32
            )
            p = jnp.exp(s * scale)
            l = jnp.sum(p, axis=-1, keepdims=True)
            o = lax.dot_general(
                p.astype(jnp.bfloat16), v, (((1,), (0,)), ((), ())),
                preferred_element_type=jnp.float32,
            )
            i = pl.ds(b * H + h, 1)

            @pl.when(c == 0)
            def _init():
                acc_o[i] = o[None]
                l_comm[0, i] = l[None]

            @pl.when(c > 0)
            def _accum():
                acc_o[i] += o[None]
                l_comm[0, i] += l[None]

        @pl.when(jnp.logical_and(b == B - 1, c == NC - 1))
        def _exchange_and_combine():
            for j in range(BH):
                o_comm[0, j] = acc_o[j].astype(jnp.bfloat16)
            rdma_o = pltpu.make_async_remote_copy(
                src_ref=o_comm.at[0],
                dst_ref=o_comm.at[1],
                send_sem=send_sems.at[0],
                recv_sem=recv_sems.at[0],
                device_id=peer,
                device_id_type=pl.DeviceIdType.MESH,
            )
            rdma_l = pltpu.make_async_remote_copy(
                src_ref=l_comm.at[0],
                dst_ref=l_comm.at[1],
                send_sem=send_sems.at[1],
                recv_sem=recv_sems.at[1],
                device_id=peer,
                device_id_type=pl.DeviceIdType.MESH,
            )
            rdma_o.start()
            rdma_l.start()
            rdma_o.wait()
            rdma_l.wait()
            for j in range(BH):
                o_tot = acc_o[j] + o_comm[1, j].astype(jnp.float32)
                l_tot = l_comm[0, j] + l_comm[1, j]
                out_ref[j] = o_tot / l_tot

    out = pl.pallas_call(
        body,
        grid=(B, NC),
        in_specs=[
            pl.BlockSpec((1, Sq, H * D), lambda b, c: (b, 0, 0)),
            pl.BlockSpec((1, C, H * D), lambda b, c: (b, c, 0)),
            pl.BlockSpec((1, C, H * D), lambda b, c: (b, c, 0)),
        ],
        out_specs=pl.BlockSpec((BH, Sq, D), lambda b, c: (0, 0, 0)),
        out_shape=jax.ShapeDtypeStruct((BH, Sq, D), jnp.float32),
        scratch_shapes=[
            pltpu.VMEM((BH, Sq, D), jnp.float32),
            pltpu.VMEM((2, BH, Sq, D), jnp.bfloat16),
            pltpu.VMEM((2, BH, Sq, 1), jnp.float32),
            pltpu.SemaphoreType.DMA((2,)),
            pltpu.SemaphoreType.DMA((2,)),
        ],
        compiler_params=pltpu.CompilerParams(
            dimension_semantics=("arbitrary", "arbitrary"),
            collective_id=0,
        ),
    )(Q.reshape(B, Sq, H * D), K.reshape(B, Skv, H * D), V.reshape(B, Skv, H * D))
    return out.reshape(B, H, Sq, D).transpose(0, 2, 1, 3)


# device time: 192118 ns/iter; 1.2213x vs baseline; 1.2213x over previous
import jax
import jax.numpy as jnp
from jax import lax
from jax.experimental import pallas as pl
from jax.experimental.pallas import tpu as pltpu

C = 512


def kernel(Q, K, V):
    B, Sq, H, D = Q.shape
    Skv = K.shape[1]
    BH = B * H
    NC = Skv // C
    scale = D ** -0.5

    def body(q_ref, k_ref, v_ref, out_ref,
             acc_o, o_comm, l_comm, send_sems, recv_sems):
        b = pl.program_id(0)
        c = pl.program_id(1)
        x = lax.axis_index("x")
        y = lax.axis_index("y")
        z = lax.axis_index("z")
        peer = (x, y, 1 - z)

        @pl.when(jnp.logical_and(b == 0, c == 0))
        def _entry_barrier():
            bsem = pltpu.get_barrier_semaphore()
            pl.semaphore_signal(
                bsem, inc=1, device_id=peer, device_id_type=pl.DeviceIdType.MESH
            )
            pl.semaphore_wait(bsem, 1)

        for h in range(H):
            q = q_ref[0, :, h, :].astype(jnp.bfloat16)
            k = k_ref[0, :, h, :].astype(jnp.bfloat16)
            v = v_ref[0, :, h, :].astype(jnp.bfloat16)
            s = lax.dot_general(
                q, k, (((1,), (1,)), ((), ())), preferred_element_type=jnp.float32
            )
            p = jnp.exp(s * scale)
            l = jnp.sum(p, axis=-1, keepdims=True)
            o = lax.dot_general(
                p.astype(jnp.bfloat16), v, (((1,), (0,)), ((), ())),
                preferred_element_type=jnp.float32,
            )
            i = pl.ds(b * H + h, 1)

            @pl.when(c == 0)
            def _init():
                acc_o[i] = o[None]
                l_comm[0, i] = l[None]

            @pl.when(c > 0)
            def _accum():
                acc_o[i] += o[None]
                l_comm[0, i] += l[None]

        @pl.when(jnp.logical_and(b == B - 1, c == NC - 1))
        def _exchange_and_combine():
            for j in range(BH):
                o_comm[0, j] = acc_o[j].astype(jnp.bfloat16)
            rdma_o = pltpu.make_async_remote_copy(
                src_ref=o_comm.at[0],
                dst_ref=o_comm.at[1],
                send_sem=send_sems.at[0],
                recv_sem=recv_sems.at[0],
                device_id=peer,
                device_id_type=pl.DeviceIdType.MESH,
            )
            rdma_l = pltpu.make_async_remote_copy(
                src_ref=l_comm.at[0],
                dst_ref=l_comm.at[1],
                send_sem=send_sems.at[1],
                recv_sem=recv_sems.at[1],
                device_id=peer,
                device_id_type=pl.DeviceIdType.MESH,
            )
            rdma_o.start()
            rdma_l.start()
            rdma_o.wait()
            rdma_l.wait()
            for j in range(BH):
                o_tot = acc_o[j] + o_comm[1, j].astype(jnp.float32)
                l_tot = l_comm[0, j] + l_comm[1, j]
                out_ref[j] = o_tot / l_tot

    out = pl.pallas_call(
        body,
        grid=(B, NC),
        in_specs=[
            pl.BlockSpec((1, Sq, H, D), lambda b, c: (b, 0, 0, 0)),
            pl.BlockSpec((1, C, H, D), lambda b, c: (b, c, 0, 0)),
            pl.BlockSpec((1, C, H, D), lambda b, c: (b, c, 0, 0)),
        ],
        out_specs=pl.BlockSpec((BH, Sq, D), lambda b, c: (0, 0, 0)),
        out_shape=jax.ShapeDtypeStruct((BH, Sq, D), jnp.float32),
        scratch_shapes=[
            pltpu.VMEM((BH, Sq, D), jnp.float32),
            pltpu.VMEM((2, BH, Sq, D), jnp.bfloat16),
            pltpu.VMEM((2, BH, Sq, 1), jnp.float32),
            pltpu.SemaphoreType.DMA((2,)),
            pltpu.SemaphoreType.DMA((2,)),
        ],
        compiler_params=pltpu.CompilerParams(
            dimension_semantics=("arbitrary", "arbitrary"),
            collective_id=0,
        ),
    )(Q, K, V)
    return out.reshape(B, H, Sq, D).transpose(0, 2, 1, 3)


# device time: 65388 ns/iter; 3.5883x vs baseline; 2.9381x over previous
import jax
import jax.numpy as jnp
from jax import lax
from jax.experimental import pallas as pl
from jax.experimental.pallas import tpu as pltpu

C = 1024
G = 4


def kernel(Q, K, V):
    B, Sq, H, D = Q.shape
    Skv = K.shape[1]
    BH = B * H
    NC = Skv // C
    NG = H // G
    rows, cols = Sq * G, C * G
    scale = D ** -0.5
    f32, bf16 = jnp.float32, jnp.bfloat16

    def body(q_ref, k_ref, v_ref, out_ref,
             acc_o, acc_l, o_comm, l_comm, send_sems, recv_sems):
        b = pl.program_id(0)
        c = pl.program_id(1)
        x = lax.axis_index("x")
        y = lax.axis_index("y")
        z = lax.axis_index("z")
        peer = (x, y, 1 - z)

        @pl.when(jnp.logical_and(b == 0, c == 0))
        def _entry_barrier():
            bsem = pltpu.get_barrier_semaphore()
            pl.semaphore_signal(
                bsem, inc=1, device_id=peer, device_id_type=pl.DeviceIdType.MESH
            )
            pl.semaphore_wait(bsem, 1)

        @pl.when(c == 0)
        def _zero_acc():
            acc_o[...] = jnp.zeros((NG, rows, D), f32)
            acc_l[...] = jnp.zeros((NG, rows, 1), f32)

        r = lax.broadcasted_iota(jnp.int32, (rows, cols), 0) % G
        cc = lax.broadcasted_iota(jnp.int32, (rows, cols), 1) % G
        bias = jnp.where(r == cc, 0.0, -1e9)

        for gi in range(NG):
            h0 = gi * G
            qg = q_ref[0, :, h0:h0 + G, :].astype(bf16).reshape(rows, D)
            kg = k_ref[0, :, h0:h0 + G, :].astype(bf16).reshape(cols, D)
            vg = v_ref[0, :, h0:h0 + G, :].astype(bf16).reshape(cols, D)
            s = lax.dot_general(
                qg, kg, (((1,), (1,)), ((), ())), preferred_element_type=f32
            )
            p = jnp.exp(s * scale + bias)
            l = jnp.sum(p, axis=-1, keepdims=True)
            o = lax.dot_general(
                p.astype(bf16), vg, (((1,), (0,)), ((), ())),
                preferred_element_type=f32,
            )
            acc_o[gi] += o
            acc_l[gi] += l

        @pl.when(c == NC - 1)
        def _unpack():
            for gi in range(NG):
                for h in range(G):
                    i = pl.ds(b * H + gi * G + h, 1)
                    o_comm[0, i] = acc_o[gi, h::G, :].astype(bf16)[None]
                    l_comm[0, i] = acc_l[gi, h::G, :][None]

        @pl.when(jnp.logical_and(b == B - 1, c == NC - 1))
        def _exchange_and_combine():
            rdma_o = pltpu.make_async_remote_copy(
                src_ref=o_comm.at[0],
                dst_ref=o_comm.at[1],
                send_sem=send_sems.at[0],
                recv_sem=recv_sems.at[0],
                device_id=peer,
                device_id_type=pl.DeviceIdType.MESH,
            )
            rdma_l = pltpu.make_async_remote_copy(
                src_ref=l_comm.at[0],
                dst_ref=l_comm.at[1],
                send_sem=send_sems.at[1],
                recv_sem=recv_sems.at[1],
                device_id=peer,
                device_id_type=pl.DeviceIdType.MESH,
            )
            rdma_o.start()
            rdma_l.start()
            rdma_o.wait()
            rdma_l.wait()
            for j in range(BH):
                o_tot = o_comm[0, j].astype(f32) + o_comm[1, j].astype(f32)
                l_tot = l_comm[0, j] + l_comm[1, j]
                out_ref[j] = o_tot / l_tot

    out = pl.pallas_call(
        body,
        grid=(B, NC),
        in_specs=[
            pl.BlockSpec((1, Sq, H, D), lambda b, c: (b, 0, 0, 0)),
            pl.BlockSpec((1, C, H, D), lambda b, c: (b, c, 0, 0)),
            pl.BlockSpec((1, C, H, D), lambda b, c: (b, c, 0, 0)),
        ],
        out_specs=pl.BlockSpec((BH, Sq, D), lambda b, c: (0, 0, 0)),
        out_shape=jax.ShapeDtypeStruct((BH, Sq, D), jnp.float32),
        scratch_shapes=[
            pltpu.VMEM((NG, rows, D), jnp.float32),
            pltpu.VMEM((NG, rows, 1), jnp.float32),
            pltpu.VMEM((2, BH, Sq, D), jnp.bfloat16),
            pltpu.VMEM((2, BH, Sq, 1), jnp.float32),
            pltpu.SemaphoreType.DMA((2,)),
            pltpu.SemaphoreType.DMA((2,)),
        ],
        compiler_params=pltpu.CompilerParams(
            dimension_semantics=("arbitrary", "arbitrary"),
            collective_id=0,
        ),
    )(Q, K, V)
    return out.reshape(B, H, Sq, D).transpose(0, 2, 1, 3)


# device time: 58952 ns/iter; 3.9801x vs baseline; 1.1092x over previous
import jax
import jax.numpy as jnp
from jax import lax
from jax.experimental import pallas as pl
from jax.experimental.pallas import tpu as pltpu

C = 1024
G = 4


def kernel(Q, K, V):
    B, Sq, H, D = Q.shape
    Skv = K.shape[1]
    BH = B * H
    NC = Skv // C
    NG = H // G
    rows, cols = Sq * G, C * G
    scale = D ** -0.5
    f32, bf16 = jnp.float32, jnp.bfloat16

    def body(q_ref, k_ref, v_ref, out_ref,
             acc_o, acc_l, o_comm, l_comm, send_sems, recv_sems):
        b = pl.program_id(0)
        c = pl.program_id(1)
        x = lax.axis_index("x")
        y = lax.axis_index("y")
        z = lax.axis_index("z")
        peer = (x, y, 1 - z)

        @pl.when(jnp.logical_and(b == 0, c == 0))
        def _entry_barrier():
            bsem = pltpu.get_barrier_semaphore()
            pl.semaphore_signal(
                bsem, inc=1, device_id=peer, device_id_type=pl.DeviceIdType.MESH
            )
            pl.semaphore_wait(bsem, 1)

        @pl.when(c == 0)
        def _zero_acc():
            acc_o[...] = jnp.zeros((NG, rows, D), f32)
            acc_l[...] = jnp.zeros((NG, rows, 1), f32)

        r = lax.broadcasted_iota(jnp.int32, (rows, cols), 0) % G
        cc = lax.broadcasted_iota(jnp.int32, (rows, cols), 1) % G
        bias = jnp.where(r == cc, 0.0, -1e9)

        for gi in range(NG):
            h0 = gi * G
            qg = q_ref[0, :, h0:h0 + G, :].astype(bf16).reshape(rows, D)
            kg = k_ref[0, :, h0:h0 + G, :].astype(bf16).reshape(cols, D)
            vg = v_ref[0, :, h0:h0 + G, :].astype(bf16).reshape(cols, D)
            s = lax.dot_general(
                qg, kg, (((1,), (1,)), ((), ())), preferred_element_type=f32
            )
            p = jnp.exp(s * scale + bias)
            l = jnp.sum(p, axis=-1, keepdims=True)
            o = lax.dot_general(
                p.astype(bf16), vg, (((1,), (0,)), ((), ())),
                preferred_element_type=f32,
            )
            acc_o[gi] += o
            acc_l[gi] += l

        def rdmas(bb):
            row = pl.ds(bb * H, H)
            rdma_o = pltpu.make_async_remote_copy(
                src_ref=o_comm.at[0, row],
                dst_ref=o_comm.at[1, row],
                send_sem=send_sems.at[0, bb],
                recv_sem=recv_sems.at[0, bb],
                device_id=peer,
                device_id_type=pl.DeviceIdType.MESH,
            )
            rdma_l = pltpu.make_async_remote_copy(
                src_ref=l_comm.at[0, row],
                dst_ref=l_comm.at[1, row],
                send_sem=send_sems.at[1, bb],
                recv_sem=recv_sems.at[1, bb],
                device_id=peer,
                device_id_type=pl.DeviceIdType.MESH,
            )
            return rdma_o, rdma_l

        for bb in range(B):
            @pl.when(jnp.logical_and(c == NC - 1, b == bb))
            def _unpack_and_send(bb=bb):
                for gi in range(NG):
                    for h in range(G):
                        j = bb * H + gi * G + h
                        o_comm[0, j] = acc_o[gi, h::G, :].astype(bf16)
                        l_comm[0, j] = acc_l[gi, h::G, :]
                rdma_o, rdma_l = rdmas(bb)
                rdma_o.start()
                rdma_l.start()

        @pl.when(jnp.logical_and(b == B - 1, c == NC - 1))
        def _wait_and_combine():
            for bb in range(B):
                rdma_o, rdma_l = rdmas(bb)
                rdma_o.wait_send()
                rdma_l.wait_send()
            for bb in range(B):
                rdma_o, rdma_l = rdmas(bb)
                rdma_o.wait_recv()
                rdma_l.wait_recv()
            o_tot = o_comm[0].astype(f32) + o_comm[1].astype(f32)
            l_tot = l_comm[0] + l_comm[1]
            out_ref[...] = o_tot / l_tot

    out = pl.pallas_call(
        body,
        grid=(B, NC),
        in_specs=[
            pl.BlockSpec((1, Sq, H, D), lambda b, c: (b, 0, 0, 0)),
            pl.BlockSpec((1, C, H, D), lambda b, c: (b, c, 0, 0)),
            pl.BlockSpec((1, C, H, D), lambda b, c: (b, c, 0, 0)),
        ],
        out_specs=pl.BlockSpec((BH, Sq, D), lambda b, c: (0, 0, 0)),
        out_shape=jax.ShapeDtypeStruct((BH, Sq, D), jnp.float32),
        scratch_shapes=[
            pltpu.VMEM((NG, rows, D), jnp.float32),
            pltpu.VMEM((NG, rows, 1), jnp.float32),
            pltpu.VMEM((2, BH, Sq, D), jnp.bfloat16),
            pltpu.VMEM((2, BH, Sq, 1), jnp.float32),
            pltpu.SemaphoreType.DMA((2, B)),
            pltpu.SemaphoreType.DMA((2, B)),
        ],
        compiler_params=pltpu.CompilerParams(
            dimension_semantics=("arbitrary", "arbitrary"),
            collective_id=0,
        ),
    )(Q, K, V)
    return out.reshape(B, H, Sq, D).transpose(0, 2, 1, 3)
